# baseline (speedup 1.0000x reference)
import functools

import jax
import jax.numpy as jnp
from jax import lax
from jax.experimental import pallas as pl
from jax.experimental.pallas import tpu as pltpu
from jax.experimental.pallas import tpu_sc as plsc

N_NODES = 10000
N_EDGES = 160000
F_IN = 128
F_EDGE = 16
HID = 16
N_CORES = 2
N_SUB = 16
NW = N_CORES * N_SUB
CHUNK = 128
NCHUNK = 40
EPW = NCHUNK * CHUNK
E_PAD = NW * EPW

@functools.lru_cache(maxsize=None)
def _mesh():
    return plsc.VectorSubcoreMesh(
        core_axis_name="c", subcore_axis_name="s",
        num_cores=N_CORES, num_subcores=N_SUB,
    )


@functools.lru_cache(maxsize=None)
def _make_gather(d):
    @functools.partial(
        pl.kernel,
        out_type=jax.ShapeDtypeStruct((E_PAD, d), jnp.float32),
        mesh=_mesh(),
        scratch_types=[
            pltpu.VMEM((NCHUNK, CHUNK), jnp.int32),
            pltpu.VMEM((CHUNK, d), jnp.float32),
            pltpu.SemaphoreType.DMA,
        ],
    )
    def gather_k(table, idx, out, idx_v, rows_v, sem):
        wid = lax.axis_index("s") * N_CORES + lax.axis_index("c")
        pltpu.sync_copy(idx.at[wid], idx_v)
        base = wid * EPW

        def step(i, carry):
            pltpu.async_copy(table.at[idx_v.at[i]], rows_v, sem).wait()
            pltpu.sync_copy(rows_v, out.at[pl.ds(base + i * CHUNK, CHUNK)])
            return carry

        lax.fori_loop(0, NCHUNK, step, 0)

    return gather_k


@functools.lru_cache(maxsize=None)
def _make_scatter(w):
    @functools.partial(
        pl.kernel,
        out_type=jax.ShapeDtypeStruct((N_CORES, N_NODES, w), jnp.float32),
        mesh=_mesh(),
        scratch_types=[
            pltpu.VMEM((NCHUNK, CHUNK), jnp.int32),
            pltpu.VMEM((CHUNK,), jnp.int32),
            pltpu.VMEM((CHUNK, w), jnp.float32),
            pltpu.VMEM_SHARED((N_NODES, w), jnp.float32),
            pltpu.SemaphoreType.DMA,
        ],
    )
    def scatter_k(msg, idx, zeros, out, idx_v, idx_cur, rows_v, acc_sh, sem):
        cid = lax.axis_index("c")
        sid = lax.axis_index("s")
        wid = sid * N_CORES + cid

        @pl.when(sid == 0)
        def _():
            pltpu.sync_copy(zeros, acc_sh)

        plsc.subcore_barrier()
        pltpu.sync_copy(idx.at[wid], idx_v)
        base = wid * EPW

        def step(i, carry):
            pltpu.sync_copy(msg.at[pl.ds(base + i * CHUNK, CHUNK)], rows_v)
            for j in range(CHUNK // 16):
                idx_cur[pl.ds(j * 16, 16)] = idx_v[i, pl.ds(j * 16, 16)]
            pltpu.sync_copy(rows_v, acc_sh.at[idx_cur], add=True)
            return carry

        lax.fori_loop(0, NCHUNK, step, 0)
        plsc.subcore_barrier()

        @pl.when(sid == 0)
        def _():
            pltpu.sync_copy(acc_sh, out.at[cid])

    return scatter_k


_EDGE_BLK = 1024
_MSG_W = 128


def _edge_body(ea_ref, xs_ref, w1_ref, b1_ref, wext_ref, out_ref, *, with_ones):
    pid = pl.program_id(0)
    b = ea_ref.shape[0]
    row = pid * b + lax.broadcasted_iota(jnp.int32, (b, 1), 0)
    valid = (row < N_EDGES).astype(jnp.float32)
    h = jnp.maximum(
        jnp.dot(ea_ref[...], w1_ref[...], preferred_element_type=jnp.float32)
        + b1_ref[...],
        0.0,
    )
    y = jnp.dot(xs_ref[...], wext_ref[...], preferred_element_type=jnp.float32)
    msg = y[:, 400:416]
    for k in range(25):
        msg = msg + h[:, k : k + 1] * y[:, k * 16 : (k + 1) * 16]
    msg = msg * valid
    if with_ones:
        out_ref[...] = jnp.concatenate(
            [msg, valid, jnp.zeros((b, _MSG_W - 17), jnp.float32)], axis=1
        )
    else:
        out_ref[...] = jnp.concatenate(
            [msg, jnp.zeros((b, _MSG_W - 16), jnp.float32)], axis=1
        )


def _make_edge(c_in, with_ones):
    w_out = _MSG_W
    grid = E_PAD // _EDGE_BLK
    return pl.pallas_call(
        functools.partial(_edge_body, with_ones=with_ones),
        grid=(grid,),
        in_specs=[
            pl.BlockSpec((_EDGE_BLK, F_EDGE), lambda i: (i, 0)),
            pl.BlockSpec((_EDGE_BLK, c_in), lambda i: (i, 0)),
            pl.BlockSpec((F_EDGE, 25), lambda i: (0, 0)),
            pl.BlockSpec((1, 25), lambda i: (0, 0)),
            pl.BlockSpec((c_in, 416), lambda i: (0, 0)),
        ],
        out_specs=pl.BlockSpec((_EDGE_BLK, w_out), lambda i: (i, 0)),
        out_shape=jax.ShapeDtypeStruct((E_PAD, w_out), jnp.float32),
    )


_edge1 = _make_edge(F_IN, True)
_edge23 = _make_edge(F_IN, False)


def _node1_body(p_ref, x_ref, root_ref, bias_ref, h_ref, invc_ref):
    s = p_ref[0] + p_ref[1]
    invc = 1.0 / jnp.maximum(s[:, 16:17], 1.0)
    val = (
        s[:, :16] * invc
        + jnp.dot(x_ref[...], root_ref[...], preferred_element_type=jnp.float32)
        + bias_ref[...]
    )
    h_ref[...] = jnp.concatenate(
        [jnp.maximum(val, 0.0), jnp.zeros((N_NODES, F_IN - HID), jnp.float32)],
        axis=1,
    )
    invc_ref[...] = invc


_node1 = pl.pallas_call(
    _node1_body,
    out_shape=(
        jax.ShapeDtypeStruct((N_NODES, F_IN), jnp.float32),
        jax.ShapeDtypeStruct((N_NODES, 1), jnp.float32),
    ),
)


def _node23_body(p_ref, act_ref, invc_ref, root_ref, bias_ref, out_ref, *, relu):
    s = (p_ref[0] + p_ref[1])[:, :HID]
    val = (
        s * invc_ref[...]
        + jnp.dot(act_ref[...], root_ref[...], preferred_element_type=jnp.float32)
        + bias_ref[...]
    )
    if relu:
        out_ref[...] = jnp.concatenate(
            [jnp.maximum(val, 0.0), jnp.zeros((N_NODES, F_IN - HID), jnp.float32)],
            axis=1,
        )
    else:
        out_ref[...] = val


def _make_node23(relu):
    return pl.pallas_call(
        functools.partial(_node23_body, relu=relu),
        out_shape=jax.ShapeDtypeStruct(
            (N_NODES, F_IN if relu else HID), jnp.float32
        ),
    )


_node2 = _make_node23(True)
_node3 = _make_node23(False)


def _pack_wext(w2, b2, c_in):
    w = w2.reshape(25, c_in, 16).transpose(1, 0, 2).reshape(c_in, 400)
    w = jnp.concatenate([w, b2.reshape(c_in, 16)], axis=1)
    if c_in < F_IN:
        w = jnp.concatenate([w, jnp.zeros((F_IN - c_in, 416), jnp.float32)], 0)
    return w


def _pad_root(root, c_in):
    if c_in < F_IN:
        root = jnp.concatenate(
            [root, jnp.zeros((F_IN - c_in, HID), jnp.float32)], 0
        )
    return root


def kernel(x, edge_index, edge_attr,
           c1_w1, c1_b1, c1_w2, c1_b2, c1_root, c1_bias,
           c2_w1, c2_b1, c2_w2, c2_b2, c2_root, c2_bias,
           c3_w1, c3_b1, c3_w2, c3_b2, c3_root, c3_bias):
    pad = jnp.zeros((2, E_PAD - N_EDGES), jnp.int32)
    ei = jnp.concatenate([edge_index, pad], axis=1)
    src = ei[0].reshape(NW, NCHUNK, CHUNK)
    dst = ei[1].reshape(NW, NCHUNK, CHUNK)
    ea = jnp.concatenate(
        [edge_attr, jnp.zeros((E_PAD - N_EDGES, F_EDGE), jnp.float32)], axis=0
    )
    zeros = jnp.zeros((N_NODES, _MSG_W), jnp.float32)

    wext1 = _pack_wext(c1_w2, c1_b2, F_IN)
    wext2 = _pack_wext(c2_w2, c2_b2, HID)
    wext3 = _pack_wext(c3_w2, c3_b2, HID)

    xsrc = _make_gather(F_IN)(x, src)
    msg1 = _edge1(ea, xsrc, c1_w1, c1_b1.reshape(1, 25), wext1)
    p1 = _make_scatter(_MSG_W)(msg1, dst, zeros)
    h1, invc = _node1(p1, x, c1_root, c1_bias.reshape(1, HID))

    hs1 = _make_gather(F_IN)(h1, src)
    msg2 = _edge23(ea, hs1, c2_w1, c2_b1.reshape(1, 25), wext2)
    p2 = _make_scatter(_MSG_W)(msg2, dst, zeros)
    h2 = _node2(p2, h1, invc, _pad_root(c2_root, HID), c2_bias.reshape(1, HID))

    hs2 = _make_gather(F_IN)(h2, src)
    msg3 = _edge23(ea, hs2, c3_w1, c3_b1.reshape(1, 25), wext3)
    p3 = _make_scatter(_MSG_W)(msg3, dst, zeros)
    out = _node3(p3, h2, invc, _pad_root(c3_root, HID), c3_bias.reshape(1, HID))
    return out


# speedup vs baseline: 2.6798x; 2.6798x over previous
import functools

import jax
import jax.numpy as jnp
from jax import lax
from jax.experimental import pallas as pl
from jax.experimental.pallas import tpu as pltpu
from jax.experimental.pallas import tpu_sc as plsc

N_NODES = 10000
N_EDGES = 160000
F_IN = 128
F_EDGE = 16
HID = 16
N_CORES = 2
N_SUB = 16
NW = N_CORES * N_SUB
CHUNK = 128
NCHUNK = 40
EPW = NCHUNK * CHUNK
E_PAD = NW * EPW

@functools.lru_cache(maxsize=None)
def _mesh():
    return plsc.VectorSubcoreMesh(
        core_axis_name="c", subcore_axis_name="s",
        num_cores=N_CORES, num_subcores=N_SUB,
    )


@functools.lru_cache(maxsize=None)
def _make_gather(d):
    @functools.partial(
        pl.kernel,
        out_type=jax.ShapeDtypeStruct((E_PAD, d), jnp.float32),
        mesh=_mesh(),
        scratch_types=[
            pltpu.VMEM((NCHUNK, CHUNK), jnp.int32),
            pltpu.VMEM((CHUNK, d), jnp.float32),
            pltpu.VMEM((CHUNK, d), jnp.float32),
            pltpu.SemaphoreType.DMA,
            pltpu.SemaphoreType.DMA,
        ],
    )
    def gather_k(table, idx, out, idx_v, rows_a, rows_b, sg_a, sg_b):
        wid = lax.axis_index("s") * N_CORES + lax.axis_index("c")
        pltpu.sync_copy(idx.at[wid], idx_v)
        base = wid * EPW
        rows = (rows_a, rows_b)
        sg = (sg_a, sg_b)
        pltpu.async_copy(table.at[idx_v.at[0]], rows_a, sg_a)

        def step(g, carry):
            cur = lax.rem(g, 2)

            @pl.when(g + 1 < NCHUNK)
            def _():
                for bsel in range(2):
                    @pl.when(cur == bsel)
                    def _():
                        pltpu.async_copy(
                            table.at[idx_v.at[g + 1]], rows[1 - bsel], sg[1 - bsel]
                        )

            for bsel in range(2):
                @pl.when(cur == bsel)
                def _():
                    pltpu.make_async_copy(
                        table.at[idx_v.at[g]], rows[bsel], sg[bsel]
                    ).wait()
                    pltpu.sync_copy(
                        rows[bsel], out.at[pl.ds(base + g * CHUNK, CHUNK)]
                    )
            return carry

        lax.fori_loop(0, NCHUNK, step, 0)

    return gather_k


@functools.lru_cache(maxsize=None)
def _make_scatter(w):
    @functools.partial(
        pl.kernel,
        out_type=jax.ShapeDtypeStruct((N_CORES, N_NODES, w), jnp.float32),
        mesh=_mesh(),
        scratch_types=[
            pltpu.VMEM((NCHUNK, CHUNK), jnp.int32),
            pltpu.VMEM((CHUNK,), jnp.int32),
            pltpu.VMEM((CHUNK, w), jnp.float32),
            pltpu.VMEM_SHARED((N_NODES, w), jnp.float32),
            pltpu.SemaphoreType.DMA,
        ],
    )
    def scatter_k(msg, idx, zeros, out, idx_v, idx_cur, rows_v, acc_sh, sem):
        cid = lax.axis_index("c")
        sid = lax.axis_index("s")
        wid = sid * N_CORES + cid

        @pl.when(sid == 0)
        def _():
            pltpu.sync_copy(zeros, acc_sh)

        plsc.subcore_barrier()
        pltpu.sync_copy(idx.at[wid], idx_v)
        base = wid * EPW

        def step(i, carry):
            pltpu.sync_copy(msg.at[pl.ds(base + i * CHUNK, CHUNK)], rows_v)
            for j in range(CHUNK // 16):
                idx_cur[pl.ds(j * 16, 16)] = idx_v[i, pl.ds(j * 16, 16)]
            pltpu.sync_copy(rows_v, acc_sh.at[idx_cur], add=True)
            return carry

        lax.fori_loop(0, NCHUNK, step, 0)
        plsc.subcore_barrier()

        @pl.when(sid == 0)
        def _():
            pltpu.sync_copy(acc_sh, out.at[cid])

    return scatter_k


_EDGE_BLK = 1024
_MSG_W = 128


def _edge_body(ea_ref, xs_ref, w1_ref, b1_ref, wext_ref, rh_ref, f_ref,
               out_ref, *, with_ones):
    pid = pl.program_id(0)
    b = ea_ref.shape[0]
    row = pid * b + lax.broadcasted_iota(jnp.int32, (b, 1), 0)
    valid = (row < N_EDGES).astype(jnp.float32)
    h = jnp.maximum(
        jnp.dot(ea_ref[...], w1_ref[...], preferred_element_type=jnp.float32)
        + b1_ref[...],
        0.0,
    )
    hext = jnp.concatenate(
        [h, jnp.ones((b, 1), jnp.float32), jnp.zeros((b, 6), jnp.float32)], axis=1
    )
    y = jnp.dot(xs_ref[...], wext_ref[...], preferred_element_type=jnp.float32)
    hrep = jnp.dot(hext, rh_ref[...], preferred_element_type=jnp.float32)
    t = y * hrep
    t4 = t[:, 0:128] + t[:, 128:256] + t[:, 256:384] + t[:, 384:512]
    msg = jnp.dot(t4, f_ref[...], preferred_element_type=jnp.float32) * valid
    if with_ones:
        col = lax.broadcasted_iota(jnp.int32, (1, _MSG_W), 1)
        msg = msg + valid * (col == 16).astype(jnp.float32)
    out_ref[...] = msg


def _make_edge(c_in, with_ones):
    grid = E_PAD // _EDGE_BLK
    return pl.pallas_call(
        functools.partial(_edge_body, with_ones=with_ones),
        grid=(grid,),
        in_specs=[
            pl.BlockSpec((_EDGE_BLK, F_EDGE), lambda i: (i, 0)),
            pl.BlockSpec((_EDGE_BLK, c_in), lambda i: (i, 0)),
            pl.BlockSpec((F_EDGE, 25), lambda i: (0, 0)),
            pl.BlockSpec((1, 25), lambda i: (0, 0)),
            pl.BlockSpec((c_in, 512), lambda i: (0, 0)),
            pl.BlockSpec((32, 512), lambda i: (0, 0)),
            pl.BlockSpec((128, _MSG_W), lambda i: (0, 0)),
        ],
        out_specs=pl.BlockSpec((_EDGE_BLK, _MSG_W), lambda i: (i, 0)),
        out_shape=jax.ShapeDtypeStruct((E_PAD, _MSG_W), jnp.float32),
    )


_edge1 = _make_edge(F_IN, True)
_edge23 = _make_edge(F_IN, False)


def _node1_body(p_ref, x_ref, root_ref, bias_ref, h_ref, invc_ref):
    s = p_ref[0] + p_ref[1]
    invc = 1.0 / jnp.maximum(s[:, 16:17], 1.0)
    val = (
        s[:, :16] * invc
        + jnp.dot(x_ref[...], root_ref[...], preferred_element_type=jnp.float32)
        + bias_ref[...]
    )
    h_ref[...] = jnp.concatenate(
        [jnp.maximum(val, 0.0), jnp.zeros((N_NODES, F_IN - HID), jnp.float32)],
        axis=1,
    )
    invc_ref[...] = invc


_node1 = pl.pallas_call(
    _node1_body,
    out_shape=(
        jax.ShapeDtypeStruct((N_NODES, F_IN), jnp.float32),
        jax.ShapeDtypeStruct((N_NODES, 1), jnp.float32),
    ),
)


def _node23_body(p_ref, act_ref, invc_ref, root_ref, bias_ref, out_ref, *, relu):
    s = (p_ref[0] + p_ref[1])[:, :HID]
    val = (
        s * invc_ref[...]
        + jnp.dot(act_ref[...], root_ref[...], preferred_element_type=jnp.float32)
        + bias_ref[...]
    )
    if relu:
        out_ref[...] = jnp.concatenate(
            [jnp.maximum(val, 0.0), jnp.zeros((N_NODES, F_IN - HID), jnp.float32)],
            axis=1,
        )
    else:
        out_ref[...] = val


def _make_node23(relu):
    return pl.pallas_call(
        functools.partial(_node23_body, relu=relu),
        out_shape=jax.ShapeDtypeStruct(
            (N_NODES, F_IN if relu else HID), jnp.float32
        ),
    )


_node2 = _make_node23(True)
_node3 = _make_node23(False)


def _pack_wext(w2, b2, c_in):
    w = w2.reshape(25, c_in, 16).transpose(1, 0, 2).reshape(c_in, 400)
    w = jnp.concatenate(
        [w, b2.reshape(c_in, 16), jnp.zeros((c_in, 96), jnp.float32)], axis=1
    )
    if c_in < F_IN:
        w = jnp.concatenate([w, jnp.zeros((F_IN - c_in, 512), jnp.float32)], 0)
    return w


def _selectors():
    k = jnp.arange(32)[:, None]
    j = jnp.arange(512)[None, :]
    rh = (j // 16 == k).astype(jnp.float32)
    jj = jnp.arange(128)[:, None]
    oo = jnp.arange(_MSG_W)[None, :]
    f = ((jj % 16 == oo) & (oo < 16)).astype(jnp.float32)
    return rh, f


def _pad_root(root, c_in):
    if c_in < F_IN:
        root = jnp.concatenate(
            [root, jnp.zeros((F_IN - c_in, HID), jnp.float32)], 0
        )
    return root


def kernel(x, edge_index, edge_attr,
           c1_w1, c1_b1, c1_w2, c1_b2, c1_root, c1_bias,
           c2_w1, c2_b1, c2_w2, c2_b2, c2_root, c2_bias,
           c3_w1, c3_b1, c3_w2, c3_b2, c3_root, c3_bias):
    pad = jnp.zeros((2, E_PAD - N_EDGES), jnp.int32)
    ei = jnp.concatenate([edge_index, pad], axis=1)
    src = ei[0].reshape(NW, NCHUNK, CHUNK)
    dst = ei[1].reshape(NW, NCHUNK, CHUNK)
    ea = jnp.concatenate(
        [edge_attr, jnp.zeros((E_PAD - N_EDGES, F_EDGE), jnp.float32)], axis=0
    )
    zeros = jnp.zeros((N_NODES, _MSG_W), jnp.float32)

    wext1 = _pack_wext(c1_w2, c1_b2, F_IN)
    wext2 = _pack_wext(c2_w2, c2_b2, HID)
    wext3 = _pack_wext(c3_w2, c3_b2, HID)
    rh, f = _selectors()

    xsrc = _make_gather(F_IN)(x, src)
    msg1 = _edge1(ea, xsrc, c1_w1, c1_b1.reshape(1, 25), wext1, rh, f)
    p1 = _make_scatter(_MSG_W)(msg1, dst, zeros)
    h1, invc = _node1(p1, x, c1_root, c1_bias.reshape(1, HID))

    hs1 = _make_gather(F_IN)(h1, src)
    msg2 = _edge23(ea, hs1, c2_w1, c2_b1.reshape(1, 25), wext2, rh, f)
    p2 = _make_scatter(_MSG_W)(msg2, dst, zeros)
    h2 = _node2(p2, h1, invc, _pad_root(c2_root, HID), c2_bias.reshape(1, HID))

    hs2 = _make_gather(F_IN)(h2, src)
    msg3 = _edge23(ea, hs2, c3_w1, c3_b1.reshape(1, 25), wext3, rh, f)
    p3 = _make_scatter(_MSG_W)(msg3, dst, zeros)
    out = _node3(p3, h2, invc, _pad_root(c3_root, HID), c3_bias.reshape(1, HID))
    return out


# speedup vs baseline: 2.8174x; 1.0514x over previous
import functools

import jax
import jax.numpy as jnp
from jax import lax
from jax.experimental import pallas as pl
from jax.experimental.pallas import tpu as pltpu
from jax.experimental.pallas import tpu_sc as plsc

N_NODES = 10000
N_EDGES = 160000
F_IN = 128
F_EDGE = 16
HID = 16
N_CORES = 2
N_SUB = 16
NW = N_CORES * N_SUB
CHUNK = 128
NCHUNK = 40
EPW = NCHUNK * CHUNK
E_PAD = NW * EPW

@functools.lru_cache(maxsize=None)
def _mesh():
    return plsc.VectorSubcoreMesh(
        core_axis_name="c", subcore_axis_name="s",
        num_cores=N_CORES, num_subcores=N_SUB,
    )


@functools.lru_cache(maxsize=None)
def _make_gather(d):
    @functools.partial(
        pl.kernel,
        out_type=jax.ShapeDtypeStruct((E_PAD, d), jnp.float32),
        mesh=_mesh(),
        scratch_types=[
            pltpu.VMEM((NCHUNK, CHUNK), jnp.int32),
            pltpu.VMEM((CHUNK, d), jnp.float32),
            pltpu.VMEM((CHUNK, d), jnp.float32),
            pltpu.SemaphoreType.DMA,
            pltpu.SemaphoreType.DMA,
        ],
    )
    def gather_k(table, idx, out, idx_v, rows_a, rows_b, sg_a, sg_b):
        wid = lax.axis_index("c") * N_SUB + lax.axis_index("s")
        pltpu.sync_copy(idx.at[wid], idx_v)
        base = wid * EPW
        rows = (rows_a, rows_b)
        sg = (sg_a, sg_b)
        pltpu.async_copy(table.at[idx_v.at[0]], rows_a, sg_a)

        def step(g, carry):
            cur = lax.rem(g, 2)

            @pl.when(g + 1 < NCHUNK)
            def _():
                for bsel in range(2):
                    @pl.when(cur == bsel)
                    def _():
                        pltpu.async_copy(
                            table.at[idx_v.at[g + 1]], rows[1 - bsel], sg[1 - bsel]
                        )

            for bsel in range(2):
                @pl.when(cur == bsel)
                def _():
                    pltpu.make_async_copy(
                        table.at[idx_v.at[g]], rows[bsel], sg[bsel]
                    ).wait()
                    pltpu.sync_copy(
                        rows[bsel], out.at[pl.ds(base + g * CHUNK, CHUNK)]
                    )
            return carry

        lax.fori_loop(0, NCHUNK, step, 0)

    return gather_k


@functools.lru_cache(maxsize=None)
def _make_scatter(w):
    @functools.partial(
        pl.kernel,
        out_type=jax.ShapeDtypeStruct((N_CORES, N_NODES, w), jnp.float32),
        mesh=_mesh(),
        scratch_types=[
            pltpu.VMEM((NCHUNK, CHUNK), jnp.int32),
            pltpu.VMEM((CHUNK,), jnp.int32),
            pltpu.VMEM((CHUNK, w), jnp.float32),
            pltpu.VMEM_SHARED((N_NODES, w), jnp.float32),
            pltpu.SemaphoreType.DMA,
        ],
    )
    def scatter_k(msg, idx, zeros, out, idx_v, idx_cur, rows_v, acc_sh, sem):
        cid = lax.axis_index("c")
        sid = lax.axis_index("s")
        wid = cid * N_SUB + sid

        @pl.when(sid == 0)
        def _():
            pltpu.sync_copy(zeros, acc_sh)

        plsc.subcore_barrier()
        pltpu.sync_copy(idx.at[wid], idx_v)
        base = wid * EPW

        def step(i, carry):
            pltpu.sync_copy(msg.at[pl.ds(base + i * CHUNK, CHUNK)], rows_v)
            for j in range(CHUNK // 16):
                idx_cur[pl.ds(j * 16, 16)] = idx_v[i, pl.ds(j * 16, 16)]
            pltpu.sync_copy(rows_v, acc_sh.at[idx_cur], add=True)
            return carry

        lax.fori_loop(0, NCHUNK, step, 0)
        plsc.subcore_barrier()

        @pl.when(sid == 0)
        def _():
            pltpu.sync_copy(acc_sh, out.at[cid])

    return scatter_k


_EDGE_BLK = 1280
_MSG_W = 128


def _edge_body(ea_ref, xs_ref, w1_ref, b1_ref, wext_ref, rh_ref, f_ref,
               out_ref, *, with_ones):
    pid = pl.program_id(0)
    b = ea_ref.shape[0]
    row = pid * b + lax.broadcasted_iota(jnp.int32, (b, 1), 0)
    valid = (row < N_EDGES).astype(jnp.float32)
    h = jnp.maximum(
        jnp.dot(ea_ref[...], w1_ref[...], preferred_element_type=jnp.float32)
        + b1_ref[...],
        0.0,
    )
    hext = jnp.concatenate(
        [h, jnp.ones((b, 1), jnp.float32), jnp.zeros((b, 6), jnp.float32)], axis=1
    )
    y = jnp.dot(xs_ref[...], wext_ref[...], preferred_element_type=jnp.float32)
    hrep = jnp.dot(hext, rh_ref[...], preferred_element_type=jnp.float32)
    t = y * hrep
    t4 = t[:, 0:128] + t[:, 128:256] + t[:, 256:384] + t[:, 384:512]
    msg = jnp.dot(t4, f_ref[...], preferred_element_type=jnp.float32) * valid
    if with_ones:
        col = lax.broadcasted_iota(jnp.int32, (1, _MSG_W), 1)
        msg = msg + valid * (col == 16).astype(jnp.float32)
    out_ref[...] = msg


def _make_edge(c_in, with_ones):
    grid = E_PAD // _EDGE_BLK
    return pl.pallas_call(
        functools.partial(_edge_body, with_ones=with_ones),
        grid=(grid,),
        in_specs=[
            pl.BlockSpec((_EDGE_BLK, F_EDGE),
                         lambda i: (jnp.minimum(i, N_EDGES // _EDGE_BLK - 1), 0)),
            pl.BlockSpec((_EDGE_BLK, c_in), lambda i: (i, 0)),
            pl.BlockSpec((F_EDGE, 25), lambda i: (0, 0)),
            pl.BlockSpec((1, 25), lambda i: (0, 0)),
            pl.BlockSpec((c_in, 512), lambda i: (0, 0)),
            pl.BlockSpec((32, 512), lambda i: (0, 0)),
            pl.BlockSpec((128, _MSG_W), lambda i: (0, 0)),
        ],
        out_specs=pl.BlockSpec((_EDGE_BLK, _MSG_W), lambda i: (i, 0)),
        out_shape=jax.ShapeDtypeStruct((E_PAD, _MSG_W), jnp.float32),
    )


_edge1 = _make_edge(F_IN, True)
_edge23 = _make_edge(F_IN, False)


def _node1_body(p_ref, x_ref, root_ref, bias_ref, h_ref, invc_ref):
    s = p_ref[0] + p_ref[1]
    invc = 1.0 / jnp.maximum(s[:, 16:17], 1.0)
    val = (
        s[:, :16] * invc
        + jnp.dot(x_ref[...], root_ref[...], preferred_element_type=jnp.float32)
        + bias_ref[...]
    )
    h_ref[...] = jnp.concatenate(
        [jnp.maximum(val, 0.0), jnp.zeros((N_NODES, F_IN - HID), jnp.float32)],
        axis=1,
    )
    invc_ref[...] = invc


_node1 = pl.pallas_call(
    _node1_body,
    out_shape=(
        jax.ShapeDtypeStruct((N_NODES, F_IN), jnp.float32),
        jax.ShapeDtypeStruct((N_NODES, 1), jnp.float32),
    ),
)


def _node23_body(p_ref, act_ref, invc_ref, root_ref, bias_ref, out_ref, *, relu):
    s = (p_ref[0] + p_ref[1])[:, :HID]
    val = (
        s * invc_ref[...]
        + jnp.dot(act_ref[...], root_ref[...], preferred_element_type=jnp.float32)
        + bias_ref[...]
    )
    if relu:
        out_ref[...] = jnp.concatenate(
            [jnp.maximum(val, 0.0), jnp.zeros((N_NODES, F_IN - HID), jnp.float32)],
            axis=1,
        )
    else:
        out_ref[...] = val


def _make_node23(relu):
    return pl.pallas_call(
        functools.partial(_node23_body, relu=relu),
        out_shape=jax.ShapeDtypeStruct(
            (N_NODES, F_IN if relu else HID), jnp.float32
        ),
    )


_node2 = _make_node23(True)
_node3 = _make_node23(False)


def _pack_wext(w2, b2, c_in):
    w = w2.reshape(25, c_in, 16).transpose(1, 0, 2).reshape(c_in, 400)
    w = jnp.concatenate(
        [w, b2.reshape(c_in, 16), jnp.zeros((c_in, 96), jnp.float32)], axis=1
    )
    if c_in < F_IN:
        w = jnp.concatenate([w, jnp.zeros((F_IN - c_in, 512), jnp.float32)], 0)
    return w


def _selectors():
    k = jnp.arange(32)[:, None]
    j = jnp.arange(512)[None, :]
    rh = (j // 16 == k).astype(jnp.float32)
    jj = jnp.arange(128)[:, None]
    oo = jnp.arange(_MSG_W)[None, :]
    f = ((jj % 16 == oo) & (oo < 16)).astype(jnp.float32)
    return rh, f


def _pad_root(root, c_in):
    if c_in < F_IN:
        root = jnp.concatenate(
            [root, jnp.zeros((F_IN - c_in, HID), jnp.float32)], 0
        )
    return root


def kernel(x, edge_index, edge_attr,
           c1_w1, c1_b1, c1_w2, c1_b2, c1_root, c1_bias,
           c2_w1, c2_b1, c2_w2, c2_b2, c2_root, c2_bias,
           c3_w1, c3_b1, c3_w2, c3_b2, c3_root, c3_bias):
    pad = jnp.zeros((2, E_PAD - N_EDGES), jnp.int32)
    ei = jnp.concatenate([edge_index, pad], axis=1)
    src = ei[0].reshape(NW, NCHUNK, CHUNK)
    dst = ei[1].reshape(NW, NCHUNK, CHUNK)
    ea = edge_attr
    zeros = jnp.zeros((N_NODES, _MSG_W), jnp.float32)

    wext1 = _pack_wext(c1_w2, c1_b2, F_IN)
    wext2 = _pack_wext(c2_w2, c2_b2, HID)
    wext3 = _pack_wext(c3_w2, c3_b2, HID)
    rh, f = _selectors()

    xsrc = _make_gather(F_IN)(x, src)
    msg1 = _edge1(ea, xsrc, c1_w1, c1_b1.reshape(1, 25), wext1, rh, f)
    p1 = _make_scatter(_MSG_W)(msg1, dst, zeros)
    h1, invc = _node1(p1, x, c1_root, c1_bias.reshape(1, HID))

    hs1 = _make_gather(F_IN)(h1, src)
    msg2 = _edge23(ea, hs1, c2_w1, c2_b1.reshape(1, 25), wext2, rh, f)
    p2 = _make_scatter(_MSG_W)(msg2, dst, zeros)
    h2 = _node2(p2, h1, invc, _pad_root(c2_root, HID), c2_bias.reshape(1, HID))

    hs2 = _make_gather(F_IN)(h2, src)
    msg3 = _edge23(ea, hs2, c3_w1, c3_b1.reshape(1, 25), wext3, rh, f)
    p3 = _make_scatter(_MSG_W)(msg3, dst, zeros)
    out = _node3(p3, h2, invc, _pad_root(c3_root, HID), c3_bias.reshape(1, HID))
    return out
